# baseline (speedup 1.0000x reference)
import functools

import jax
import jax.numpy as jnp
from jax import lax
from jax.experimental import pallas as pl
from jax.experimental.pallas import tpu as pltpu
from jax.experimental.pallas import tpu_sc as plsc

NC = 2
NS = 16
NW = NC * NS
LANES = 16

OUT_CHUNK = 4096


def _make_gather(P, N, B):
    rows_per_w = P // NW
    n_chunks = B // OUT_CHUNK

    mesh = plsc.VectorSubcoreMesh(
        core_axis_name="c", subcore_axis_name="s",
        num_cores=NC, num_subcores=NS)

    @functools.partial(
        pl.kernel,
        out_type=jax.ShapeDtypeStruct((P, B), jnp.float32),
        mesh=mesh,
        scratch_types=[
            pltpu.VMEM((B,), jnp.int32),
            pltpu.VMEM((N,), jnp.float32),
            pltpu.VMEM((2, OUT_CHUNK), jnp.float32),
            pltpu.SemaphoreType.DMA,
            pltpu.SemaphoreType.DMA,
            pltpu.SemaphoreType.DMA,
        ],
        compiler_params=pltpu.CompilerParams(needs_layout_passes=False),
    )
    def gather_kernel(table_hbm, idx_hbm, out_hbm, idx_v, row_v, out_v,
                      wsem0, wsem1, isem):
        wid = lax.axis_index("s") * NC + lax.axis_index("c")
        wsems = (wsem0, wsem1)

        idx_cp = pltpu.async_copy(idx_hbm, idx_v, isem)
        pending = [None, None]

        for k in range(rows_per_w):
            rc = wid * rows_per_w + k
            pltpu.sync_copy(table_hbm.at[rc], row_v)
            if k == 0:
                idx_cp.wait()
            for chunk in range(n_chunks):
                buf = chunk % 2
                if pending[buf] is not None:
                    pending[buf].wait()
                    pending[buf] = None
                base = chunk * OUT_CHUNK

                def body(g, base=base, buf=buf):
                    iv = idx_v[pl.ds(base + g * LANES, LANES)]
                    out_v[buf, pl.ds(g * LANES, LANES)] = (
                        plsc.load_gather(row_v, [iv]))

                plsc.parallel_loop(0, OUT_CHUNK // LANES, unroll=8)(body)
                pending[buf] = pltpu.async_copy(
                    out_v.at[buf],
                    out_hbm.at[rc, pl.ds(base, OUT_CHUNK)],
                    wsems[buf])
        for p in pending:
            if p is not None:
                p.wait()

    return gather_kernel


def kernel(G, idx):
    R, N, C = G.shape
    B = idx.shape[0]
    table = jnp.transpose(G, (0, 2, 1)).reshape(R * C, N)
    out2 = _make_gather(R * C, N, B)(table, idx)
    return jnp.transpose(out2.reshape(R, C, B), (0, 2, 1))
